# baseline (speedup 1.0000x reference)
import functools
import jax
import jax.numpy as jnp
from jax import lax
from jax.experimental import pallas as pl
from jax.experimental.pallas import tpu as pltpu
from jax.experimental.pallas import tpu_sc as plsc

N = 10000
F_IN = 128
DIM = 32
C = 10
E = 320000
G = 64

NC, NS = 2, 16
NW = NC * NS
CHUNK = 128
CH_PER_W = -(-((E + NW * CHUNK - 1) // (NW * CHUNK)) // 8) * 8
E_PAD = NW * CHUNK * CH_PER_W
N_SH = 10240
ROWS_PER_SUB = N_SH // NS


NBUF = 4
NG = CH_PER_W // NBUF
RING = 4


S_STAGE = 624


def _sc_agg_body(y_hbm, src_hbm, dst_hbm, zeros_hbm, out_hbm,
                 src_v, dst_v, rows_v, shared, table, sg, ss):
    c = lax.axis_index("c")
    s = lax.axis_index("s")
    wid = s * NC + c

    pltpu.sync_copy(zeros_hbm.at[pl.ds(0, ROWS_PER_SUB)],
                    shared.at[pl.ds(s * ROWS_PER_SUB, ROWS_PER_SUB)])
    pltpu.sync_copy(y_hbm.at[pl.ds(s * S_STAGE, S_STAGE)],
                    table.at[pl.ds(s * S_STAGE, S_STAGE)])

    @pl.when(s == 0)
    def _():
        pltpu.sync_copy(y_hbm.at[pl.ds(NS * S_STAGE, N - NS * S_STAGE)],
                        table.at[pl.ds(NS * S_STAGE, N - NS * S_STAGE)])

    plsc.subcore_barrier()

    base = wid * CH_PER_W
    pltpu.sync_copy(src_hbm.at[pl.ds(base, CH_PER_W)], src_v)
    pltpu.sync_copy(dst_hbm.at[pl.ds(base, CH_PER_W)], dst_v)

    def fire_gathers(g, slot):
        for b in range(NBUF):
            pltpu.async_copy(table.at[src_v.at[g * NBUF + b]],
                             rows_v.at[slot].at[b], sg.at[slot])

    def drain_gathers(g, slot):
        for b in range(NBUF):
            pltpu.make_async_copy(table.at[src_v.at[g * NBUF + b]],
                                  rows_v.at[slot].at[b], sg.at[slot]).wait()

    def fire_scatters(g, slot):
        for b in range(NBUF):
            pltpu.async_copy(rows_v.at[slot].at[b],
                             shared.at[dst_v.at[g * NBUF + b]], ss.at[slot],
                             add=True)

    def drain_scatters(g, slot):
        for b in range(NBUF):
            pltpu.make_async_copy(rows_v.at[slot].at[b],
                                  shared.at[dst_v.at[g * NBUF + b]],
                                  ss.at[slot]).wait()

    fire_gathers(0, 0)
    fire_gathers(1, 1)

    def body(g, _):
        @pl.when(g >= 2)
        def _():
            drain_scatters(g - 2, lax.rem(g - 2, RING))

        @pl.when(g + 2 < NG)
        def _():
            fire_gathers(g + 2, lax.rem(g + 2, RING))

        slot = lax.rem(g, RING)
        drain_gathers(g, slot)
        fire_scatters(g, slot)
        return ()

    lax.fori_loop(0, NG, body, (), unroll=False)
    drain_scatters(NG - 2, (NG - 2) % RING)
    drain_scatters(NG - 1, (NG - 1) % RING)
    plsc.subcore_barrier()

    pltpu.sync_copy(shared.at[pl.ds(s * ROWS_PER_SUB, ROWS_PER_SUB)],
                    out_hbm.at[c].at[pl.ds(s * ROWS_PER_SUB, ROWS_PER_SUB)])


@jax.jit
def _sc_agg(y, src2d, dst2d, zeros_sh):
    mesh = plsc.VectorSubcoreMesh(core_axis_name="c", subcore_axis_name="s")
    return pl.kernel(
        _sc_agg_body,
        out_type=jax.ShapeDtypeStruct((NC, N_SH, DIM), jnp.float32),
        mesh=mesh,
        scratch_types=[
            pltpu.VMEM((CH_PER_W, CHUNK), jnp.int32),
            pltpu.VMEM((CH_PER_W, CHUNK), jnp.int32),
            pltpu.VMEM((RING, NBUF, CHUNK, DIM), jnp.float32),
            pltpu.VMEM_SHARED((N_SH, DIM), jnp.float32),
            pltpu.VMEM_SHARED((N, DIM), jnp.float32),
            pltpu.SemaphoreType.DMA((RING,)),
            pltpu.SemaphoreType.DMA((RING,)),
        ],
        compiler_params=pltpu.CompilerParams(use_tc_tiling_on_sc=False),
    )(y, src2d, dst2d, zeros_sh)


def _tc_mlp_body(y_ref, a0_ref, a1_ref, w2_ref, b1_ref, b2_ref,
                 bnw_ref, bnb_ref, bnm_ref, bnv_ref, wn_ref, out_ref):
    t = y_ref[...] + a0_ref[...] + a1_ref[...] + b1_ref[...]
    z = jnp.maximum(t, 0.0)
    z2 = jnp.dot(z, w2_ref[...], preferred_element_type=jnp.float32)
    z2 = jnp.maximum(z2 + b2_ref[...], 0.0)
    scale = bnw_ref[...] * lax.rsqrt(bnv_ref[...] + 1e-5)
    shift = bnb_ref[...] - bnm_ref[...] * scale
    h = z2 * scale + shift
    out_ref[...] = jnp.dot(h, wn_ref[...], preferred_element_type=jnp.float32)


def _tc_mlp_pool_body(y_ref, a0_ref, a1_ref, w2_ref, b1_ref, b2_ref,
                      bnw_ref, bnb_ref, bnm_ref, bnv_ref,
                      batch_ref, fc1w_ref, fc1b_ref, fc2w_ref, fc2b_ref,
                      out_ref, pooled_acc):
    i = pl.program_id(0)

    @pl.when(i == 0)
    def _():
        pooled_acc[...] = jnp.zeros_like(pooled_acc)

    t = y_ref[...] + a0_ref[...] + a1_ref[...] + b1_ref[...]
    z = jnp.maximum(t, 0.0)
    z2 = jnp.dot(z, w2_ref[...], preferred_element_type=jnp.float32)
    z2 = jnp.maximum(z2 + b2_ref[...], 0.0)
    scale = bnw_ref[...] * lax.rsqrt(bnv_ref[...] + 1e-5)
    shift = bnb_ref[...] - bnm_ref[...] * scale
    h = z2 * scale + shift

    batch = batch_ref[0, 0, :]
    gid = lax.broadcasted_iota(jnp.int32, (G, batch.shape[0]), 0)
    onehot = jnp.where(gid == batch[None, :], 1.0, 0.0)
    pooled_acc[...] += jnp.dot(onehot, h, preferred_element_type=jnp.float32)

    @pl.when(i == pl.num_programs(0) - 1)
    def _():
        g1 = jnp.dot(pooled_acc[...], fc1w_ref[...],
                     preferred_element_type=jnp.float32)
        g1 = jnp.maximum(g1 + fc1b_ref[...], 0.0)
        out_ref[...] = (jnp.dot(g1, fc2w_ref[...],
                                preferred_element_type=jnp.float32)
                        + fc2b_ref[...])


def _tc_x_w1_body(x_ref, w_ref, out_ref):
    out_ref[...] = jnp.dot(x_ref[...], w_ref[...],
                           preferred_element_type=jnp.float32)


B_ROWS = 2000
N_BLOCKS = N // B_ROWS

_row_spec = pl.BlockSpec((B_ROWS, DIM), lambda i: (i, 0))
_full = lambda shape: pl.BlockSpec(shape, lambda i: tuple(0 for _ in shape))


@jax.jit
def _tc_x_w1(x, w1):
    return pl.pallas_call(
        _tc_x_w1_body,
        grid=(N_BLOCKS,),
        in_specs=[pl.BlockSpec((B_ROWS, F_IN), lambda i: (i, 0)),
                  _full((F_IN, DIM))],
        out_specs=_row_spec,
        out_shape=jax.ShapeDtypeStruct((N, DIM), jnp.float32),
    )(x, w1)


@jax.jit
def _tc_mlp(y, a0, a1, w2, b1, b2, bnw, bnb, bnm, bnv, wn):
    vec = _full((1, DIM))
    return pl.pallas_call(
        _tc_mlp_body,
        grid=(N_BLOCKS,),
        in_specs=[_row_spec, _row_spec, _row_spec, _full((DIM, DIM)),
                  vec, vec, vec, vec, vec, vec, _full((DIM, DIM))],
        out_specs=_row_spec,
        out_shape=jax.ShapeDtypeStruct((N, DIM), jnp.float32),
    )(y, a0, a1, w2, b1, b2, bnw, bnb, bnm, bnv, wn)


@jax.jit
def _tc_mlp_pool(y, a0, a1, w2, b1, b2, bnw, bnb, bnm, bnv,
                 batch3d, fc1w, fc1b, fc2w, fc2b):
    vec = _full((1, DIM))
    return pl.pallas_call(
        _tc_mlp_pool_body,
        grid=(N_BLOCKS,),
        in_specs=[_row_spec, _row_spec, _row_spec, _full((DIM, DIM)),
                  vec, vec, vec, vec, vec, vec,
                  pl.BlockSpec((1, 1, B_ROWS), lambda i: (i, 0, 0)),
                  _full((DIM, DIM)), _full((1, DIM)),
                  _full((DIM, C)), _full((1, C))],
        out_specs=_full((G, C)),
        out_shape=jax.ShapeDtypeStruct((G, C), jnp.float32),
        scratch_shapes=[pltpu.VMEM((G, DIM), jnp.float32)],
    )(y, a0, a1, w2, b1, b2, bnw, bnb, bnm, bnv,
      batch3d, fc1w, fc1b, fc2w, fc2b)


def kernel(x, edge_index, batch,
           conv1_W1, conv1_b1, conv1_W2, conv1_b2,
           bn1_w, bn1_b, bn1_mean, bn1_var,
           conv2_W1, conv2_b1, conv2_W2, conv2_b2,
           bn2_w, bn2_b, bn2_mean, bn2_var,
           conv3_W1, conv3_b1, conv3_W2, conv3_b2,
           bn3_w, bn3_b, bn3_mean, bn3_var,
           conv4_W1, conv4_b1, conv4_W2, conv4_b2,
           bn4_w, bn4_b, bn4_mean, bn4_var,
           conv5_W1, conv5_b1, conv5_W2, conv5_b2,
           bn5_w, bn5_b, bn5_mean, bn5_var,
           fc1_W, fc1_b, fc2_W, fc2_b):
    p = locals()

    n_dummy = E_PAD - E
    pad = jnp.stack([jnp.zeros((n_dummy,), jnp.int32),
                     jnp.full((n_dummy,), N, jnp.int32)])
    epad = jnp.concatenate([edge_index, pad], axis=1)
    epad = epad.reshape(2, NW * CH_PER_W, CHUNK)
    src2d, dst2d = epad[0], epad[1]
    zeros_sh = jnp.zeros((ROWS_PER_SUB, DIM), jnp.float32)
    batch3d = batch.reshape(N_BLOCKS, 1, B_ROWS)

    y = _tc_x_w1(x, conv1_W1)
    for i in range(1, 6):
        agg = _sc_agg(y, src2d, dst2d, zeros_sh)
        row = lambda k: p[k].reshape(1, -1)
        if i < 5:
            y = _tc_mlp(y, agg[0], agg[1], p[f"conv{i}_W2"],
                        row(f"conv{i}_b1"), row(f"conv{i}_b2"),
                        row(f"bn{i}_w"), row(f"bn{i}_b"),
                        row(f"bn{i}_mean"), row(f"bn{i}_var"),
                        p[f"conv{i + 1}_W1"])
        else:
            out = _tc_mlp_pool(y, agg[0], agg[1], p[f"conv{i}_W2"],
                               row(f"conv{i}_b1"), row(f"conv{i}_b2"),
                               row(f"bn{i}_w"), row(f"bn{i}_b"),
                               row(f"bn{i}_mean"), row(f"bn{i}_var"),
                               batch3d, fc1_W, fc1_b.reshape(1, -1),
                               fc2_W, fc2_b.reshape(1, -1))
    return out


# speedup vs baseline: 1.2687x; 1.2687x over previous
import functools
import jax
import jax.numpy as jnp
from jax import lax
from jax.experimental import pallas as pl
from jax.experimental.pallas import tpu as pltpu
from jax.experimental.pallas import tpu_sc as plsc

N = 10000
F_IN = 128
DIM = 32
C = 10
E = 320000
G = 64

NC, NS = 2, 16
NW = NC * NS
CHUNK = 128
CH_PER_W = -(-((E + NW * CHUNK - 1) // (NW * CHUNK)) // 8) * 8
E_PAD = NW * CHUNK * CH_PER_W
NJ = 4
N_SH = 10240
PK = N_SH // NJ
ROWS_PER_SUB = N_SH // NS
SROWS = PK // NS

NBUF = 4
NG = CH_PER_W // NBUF
RING = 4


def _sc_agg_body(y_hbm, src_hbm, dst_hbm, zeros_hbm, out_hbm,
                 src_v, dst_v, rows_v, shared, table, sg, ss):
    c = lax.axis_index("c")
    s = lax.axis_index("s")
    wid = s * NC + c

    pltpu.sync_copy(zeros_hbm.at[pl.ds(0, ROWS_PER_SUB)],
                    shared.at[pl.ds(s * ROWS_PER_SUB, ROWS_PER_SUB)])
    for j in range(NJ):
        pltpu.sync_copy(
            y_hbm.at[pl.ds(s * SROWS, SROWS), pl.ds(j * DIM, DIM)],
            table.at[pl.ds(j * PK + s * SROWS, SROWS)])

    plsc.subcore_barrier()

    base = wid * CH_PER_W
    pltpu.sync_copy(src_hbm.at[pl.ds(base, CH_PER_W)], src_v)
    pltpu.sync_copy(dst_hbm.at[pl.ds(base, CH_PER_W)], dst_v)

    def fire_gathers(g, slot):
        for b in range(NBUF):
            pltpu.async_copy(table.at[src_v.at[g * NBUF + b]],
                             rows_v.at[slot].at[b], sg.at[slot])

    def drain_gathers(g, slot):
        for b in range(NBUF):
            pltpu.make_async_copy(table.at[src_v.at[g * NBUF + b]],
                                  rows_v.at[slot].at[b], sg.at[slot]).wait()

    def fire_scatters(g, slot):
        for b in range(NBUF):
            pltpu.async_copy(rows_v.at[slot].at[b],
                             shared.at[dst_v.at[g * NBUF + b]], ss.at[slot],
                             add=True)

    def drain_scatters(g, slot):
        for b in range(NBUF):
            pltpu.make_async_copy(rows_v.at[slot].at[b],
                                  shared.at[dst_v.at[g * NBUF + b]],
                                  ss.at[slot]).wait()

    fire_gathers(0, 0)
    fire_gathers(1, 1)

    def body(g, _):
        @pl.when(g >= 2)
        def _():
            drain_scatters(g - 2, lax.rem(g - 2, RING))

        @pl.when(g + 2 < NG)
        def _():
            fire_gathers(g + 2, lax.rem(g + 2, RING))

        slot = lax.rem(g, RING)
        drain_gathers(g, slot)
        fire_scatters(g, slot)
        return ()

    lax.fori_loop(0, NG, body, (), unroll=False)
    drain_scatters(NG - 2, (NG - 2) % RING)
    drain_scatters(NG - 1, (NG - 1) % RING)
    plsc.subcore_barrier()

    for j in range(NJ):
        pltpu.sync_copy(
            shared.at[pl.ds(j * PK + s * SROWS, SROWS)],
            out_hbm.at[c].at[pl.ds(s * SROWS, SROWS), pl.ds(j * DIM, DIM)])


@jax.jit
def _sc_agg(y, src2d, dst2d, zeros_sh):
    mesh = plsc.VectorSubcoreMesh(core_axis_name="c", subcore_axis_name="s")
    return pl.kernel(
        _sc_agg_body,
        out_type=jax.ShapeDtypeStruct((NC, PK, NJ * DIM), jnp.float32),
        mesh=mesh,
        scratch_types=[
            pltpu.VMEM((CH_PER_W, CHUNK), jnp.int32),
            pltpu.VMEM((CH_PER_W, CHUNK), jnp.int32),
            pltpu.VMEM((RING, NBUF, CHUNK, DIM), jnp.float32),
            pltpu.VMEM_SHARED((N_SH, DIM), jnp.float32),
            pltpu.VMEM_SHARED((N_SH, DIM), jnp.float32),
            pltpu.SemaphoreType.DMA((RING,)),
            pltpu.SemaphoreType.DMA((RING,)),
        ],
        compiler_params=pltpu.CompilerParams(use_tc_tiling_on_sc=False),
    )(y, src2d, dst2d, zeros_sh)


BP = 512
NBLK = PK // BP


def _tc_x_w1_body(x0_ref, x1_ref, x2_ref, x3_ref, w_ref, out_ref):
    i = pl.program_id(0)
    rid = lax.broadcasted_iota(jnp.int32, (BP, DIM), 0)
    for j, x_ref in enumerate((x0_ref, x1_ref, x2_ref, x3_ref)):
        yj = jnp.dot(x_ref[...], w_ref[...], preferred_element_type=jnp.float32)
        node = j * PK + i * BP + rid
        yj = jnp.where(node < N, yj, 0.0)
        out_ref[:, j * DIM:(j + 1) * DIM] = yj


def _tc_mlp_body(y_ref, a0_ref, a1_ref, w2_ref, b1_ref, b2_ref,
                 scale_ref, shift_ref, wn_ref, out_ref):
    t = y_ref[...] + a0_ref[...] + a1_ref[...] + b1_ref[...]
    z = jnp.maximum(t, 0.0)
    z2 = jnp.dot(z, w2_ref[...], preferred_element_type=jnp.float32)
    z2 = jnp.maximum(z2 + b2_ref[...], 0.0)
    h = z2 * scale_ref[...] + shift_ref[...]
    out_ref[...] = jnp.dot(h, wn_ref[...], preferred_element_type=jnp.float32)


def _tc_mlp_pool_body(y_ref, a0_ref, a1_ref, w2_ref, b1_ref, b2_ref,
                      scale_ref, shift_ref,
                      bt0_ref, bt1_ref, bt2_ref, bt3_ref,
                      fc1w_ref, fc1b_ref, fc2w_ref, fc2b_ref,
                      out_ref, pooled_acc):
    i = pl.program_id(0)

    @pl.when(i == 0)
    def _():
        pooled_acc[...] = jnp.zeros_like(pooled_acc)

    t = y_ref[...] + a0_ref[...] + a1_ref[...] + b1_ref[...]
    z = jnp.maximum(t, 0.0)
    z2 = jnp.dot(z, w2_ref[...], preferred_element_type=jnp.float32)
    z2 = jnp.maximum(z2 + b2_ref[...], 0.0)
    h = z2 * scale_ref[...] + shift_ref[...]

    gid = lax.broadcasted_iota(jnp.int32, (G, BP), 0)
    for j, bt_ref in enumerate((bt0_ref, bt1_ref, bt2_ref, bt3_ref)):
        bj = bt_ref[0, 0, :]
        onehot = jnp.where(gid == bj[None, :], 1.0, 0.0)
        pooled_acc[...] += jnp.dot(onehot, h[:, j * DIM:(j + 1) * DIM],
                                   preferred_element_type=jnp.float32)

    @pl.when(i == pl.num_programs(0) - 1)
    def _():
        g1 = jnp.dot(pooled_acc[...], fc1w_ref[...],
                     preferred_element_type=jnp.float32)
        g1 = jnp.maximum(g1 + fc1b_ref[...], 0.0)
        out_ref[...] = (jnp.dot(g1, fc2w_ref[...],
                                preferred_element_type=jnp.float32)
                        + fc2b_ref[...])


_row_spec = pl.BlockSpec((BP, NJ * DIM), lambda i: (i, 0))
_full = lambda shape: pl.BlockSpec(shape, lambda i: tuple(0 for _ in shape))


@jax.jit
def _tc_x_w1(x, w1):
    x_specs = [pl.BlockSpec((BP, F_IN), functools.partial(
        lambda j, i: (j * NBLK + i, 0), j)) for j in range(NJ)]
    return pl.pallas_call(
        _tc_x_w1_body,
        grid=(NBLK,),
        in_specs=x_specs + [_full((F_IN, DIM))],
        out_specs=_row_spec,
        out_shape=jax.ShapeDtypeStruct((PK, NJ * DIM), jnp.float32),
    )(x, x, x, x, w1)


@jax.jit
def _tc_mlp(y, a0, a1, w2, b1, b2, scale, shift, wn):
    vec = _full((1, NJ * DIM))
    bd = _full((NJ * DIM, NJ * DIM))
    return pl.pallas_call(
        _tc_mlp_body,
        grid=(NBLK,),
        in_specs=[_row_spec, _row_spec, _row_spec, bd, vec, vec, vec, vec, bd],
        out_specs=_row_spec,
        out_shape=jax.ShapeDtypeStruct((PK, NJ * DIM), jnp.float32),
    )(y, a0, a1, w2, b1, b2, scale, shift, wn)


@jax.jit
def _tc_mlp_pool(y, a0, a1, w2, b1, b2, scale, shift,
                 batch3d, fc1w, fc1b, fc2w, fc2b):
    vec = _full((1, NJ * DIM))
    bd = _full((NJ * DIM, NJ * DIM))
    bt_specs = [pl.BlockSpec((1, 1, BP), functools.partial(
        lambda j, i: (j * NBLK + i, 0, 0), j)) for j in range(NJ)]
    return pl.pallas_call(
        _tc_mlp_pool_body,
        grid=(NBLK,),
        in_specs=([_row_spec, _row_spec, _row_spec, bd, vec, vec, vec, vec]
                  + bt_specs
                  + [_full((DIM, DIM)), _full((1, DIM)),
                     _full((DIM, C)), _full((1, C))]),
        out_specs=_full((G, C)),
        out_shape=jax.ShapeDtypeStruct((G, C), jnp.float32),
        scratch_shapes=[pltpu.VMEM((G, DIM), jnp.float32)],
    )(y, a0, a1, w2, b1, b2, scale, shift,
      batch3d, batch3d, batch3d, batch3d, fc1w, fc1b, fc2w, fc2b)


def kernel(x, edge_index, batch,
           conv1_W1, conv1_b1, conv1_W2, conv1_b2,
           bn1_w, bn1_b, bn1_mean, bn1_var,
           conv2_W1, conv2_b1, conv2_W2, conv2_b2,
           bn2_w, bn2_b, bn2_mean, bn2_var,
           conv3_W1, conv3_b1, conv3_W2, conv3_b2,
           bn3_w, bn3_b, bn3_mean, bn3_var,
           conv4_W1, conv4_b1, conv4_W2, conv4_b2,
           bn4_w, bn4_b, bn4_mean, bn4_var,
           conv5_W1, conv5_b1, conv5_W2, conv5_b2,
           bn5_w, bn5_b, bn5_mean, bn5_var,
           fc1_W, fc1_b, fc2_W, fc2_b):
    p = locals()
    eye4 = jnp.eye(NJ, dtype=jnp.float32)
    bdiag = lambda w: jnp.kron(eye4, w)
    tile4 = lambda v: jnp.tile(v, NJ).reshape(1, NJ * DIM)

    n_dummy = E_PAD - E
    pad = jnp.stack([jnp.zeros((n_dummy,), jnp.int32),
                     jnp.full((n_dummy,), N, jnp.int32)])
    epad = jnp.concatenate([edge_index, pad], axis=1)
    epad = epad.reshape(2, NW * CH_PER_W, CHUNK)
    src2d, dst2d = epad[0], epad[1]
    zeros_sh = jnp.zeros((ROWS_PER_SUB, DIM), jnp.float32)
    batch_pk = jnp.concatenate([batch, jnp.full((N_SH - N,), G, jnp.int32)])
    batch3d = batch_pk.reshape(NJ * NBLK, 1, BP)

    y = _tc_x_w1(x, conv1_W1)
    for i in range(1, 6):
        agg = _sc_agg(y, src2d, dst2d, zeros_sh)
        scale = p[f"bn{i}_w"] * lax.rsqrt(p[f"bn{i}_var"] + 1e-5)
        shift = p[f"bn{i}_b"] - p[f"bn{i}_mean"] * scale
        args = (y, agg[0], agg[1], bdiag(p[f"conv{i}_W2"]),
                tile4(p[f"conv{i}_b1"]), tile4(p[f"conv{i}_b2"]),
                tile4(scale), tile4(shift))
        if i < 5:
            y = _tc_mlp(*args, bdiag(p[f"conv{i + 1}_W1"]))
        else:
            out = _tc_mlp_pool(*args, batch3d, fc1_W, fc1_b.reshape(1, -1),
                               fc2_W, fc2_b.reshape(1, -1))
    return out


# speedup vs baseline: 1.2997x; 1.0245x over previous
import functools
import jax
import jax.numpy as jnp
from jax import lax
from jax.experimental import pallas as pl
from jax.experimental.pallas import tpu as pltpu
from jax.experimental.pallas import tpu_sc as plsc

N = 10000
F_IN = 128
DIM = 32
C = 10
E = 320000
G = 64

NC, NS = 2, 16
NW = NC * NS
CHUNK = 128
CH_PER_W = -(-((E + NW * CHUNK - 1) // (NW * CHUNK)) // 8) * 8
E_PAD = NW * CHUNK * CH_PER_W
NJ = 4
N_SH = 10240
PK = N_SH // NJ
ROWS_PER_SUB = N_SH // NS
SROWS = PK // NS

NBUF = 4
NG = CH_PER_W // NBUF
RING = 4


def _sc_agg_body(y_hbm, src_hbm, dst_hbm, zeros_hbm, out_hbm,
                 src_v, dst_v, rows_v, shared, table, sg, ss):
    c = lax.axis_index("c")
    s = lax.axis_index("s")
    wid = s * NC + c

    pltpu.sync_copy(zeros_hbm.at[pl.ds(0, ROWS_PER_SUB)],
                    shared.at[pl.ds(s * ROWS_PER_SUB, ROWS_PER_SUB)])
    for j in range(NJ):
        pltpu.sync_copy(
            y_hbm.at[pl.ds(s * SROWS, SROWS), pl.ds(j * DIM, DIM)],
            table.at[pl.ds(j * PK + s * SROWS, SROWS)])

    plsc.subcore_barrier()

    base = wid * CH_PER_W
    pltpu.sync_copy(src_hbm.at[pl.ds(base, CH_PER_W)], src_v)
    pltpu.sync_copy(dst_hbm.at[pl.ds(base, CH_PER_W)], dst_v)

    def fire_gathers(g, slot):
        for b in range(NBUF):
            pltpu.async_copy(table.at[src_v.at[g * NBUF + b]],
                             rows_v.at[slot].at[b], sg.at[slot])

    def drain_gathers(g, slot):
        for b in range(NBUF):
            pltpu.make_async_copy(table.at[src_v.at[g * NBUF + b]],
                                  rows_v.at[slot].at[b], sg.at[slot]).wait()

    def fire_scatters(g, slot):
        for b in range(NBUF):
            pltpu.async_copy(rows_v.at[slot].at[b],
                             shared.at[dst_v.at[g * NBUF + b]], ss.at[slot],
                             add=True)

    def drain_scatters(g, slot):
        for b in range(NBUF):
            pltpu.make_async_copy(rows_v.at[slot].at[b],
                                  shared.at[dst_v.at[g * NBUF + b]],
                                  ss.at[slot]).wait()

    fire_gathers(0, 0)
    fire_gathers(1, 1)

    def body(g, _):
        @pl.when(g >= 2)
        def _():
            drain_scatters(g - 2, lax.rem(g - 2, RING))

        @pl.when(g + 2 < NG)
        def _():
            fire_gathers(g + 2, lax.rem(g + 2, RING))

        slot = lax.rem(g, RING)
        drain_gathers(g, slot)
        fire_scatters(g, slot)
        return ()

    lax.fori_loop(0, NG, body, (), unroll=False)
    drain_scatters(NG - 2, (NG - 2) % RING)
    drain_scatters(NG - 1, (NG - 1) % RING)
    plsc.subcore_barrier()

    for j in range(NJ):
        pltpu.sync_copy(
            shared.at[pl.ds(j * PK + s * SROWS, SROWS)],
            out_hbm.at[c].at[pl.ds(s * SROWS, SROWS), pl.ds(j * DIM, DIM)])


@jax.jit
def _sc_agg(y, src2d, dst2d, zeros_sh):
    mesh = plsc.VectorSubcoreMesh(core_axis_name="c", subcore_axis_name="s")
    return pl.kernel(
        _sc_agg_body,
        out_type=jax.ShapeDtypeStruct((NC, PK, NJ * DIM), jnp.float32),
        mesh=mesh,
        scratch_types=[
            pltpu.VMEM((CH_PER_W, CHUNK), jnp.int32),
            pltpu.VMEM((CH_PER_W, CHUNK), jnp.int32),
            pltpu.VMEM((RING, NBUF, CHUNK, DIM), jnp.float32),
            pltpu.VMEM_SHARED((N_SH, DIM), jnp.float32),
            pltpu.VMEM_SHARED((N_SH, DIM), jnp.float32),
            pltpu.SemaphoreType.DMA((RING,)),
            pltpu.SemaphoreType.DMA((RING,)),
        ],
        compiler_params=pltpu.CompilerParams(use_tc_tiling_on_sc=False),
    )(y, src2d, dst2d, zeros_sh)


BP = 512
NBLK = PK // BP


def _tc_prep_body(es_ref, ed_ref, x0_ref, x1_ref, x2_ref, x3_ref, w_ref,
                  src_ref, dst_ref, out_ref):
    i = pl.program_id(0)
    crow = i * BP + lax.broadcasted_iota(jnp.int32, (BP, CHUNK), 0)
    ev = crow < E // CHUNK
    src_ref[...] = jnp.where(ev, es_ref[0], 0)
    dst_ref[...] = jnp.where(ev, ed_ref[0], N)

    rid = lax.broadcasted_iota(jnp.int32, (BP, DIM), 0)
    for j, x_ref in enumerate((x0_ref, x1_ref, x2_ref, x3_ref)):
        yj = jnp.dot(x_ref[...], w_ref[...], preferred_element_type=jnp.float32)
        node = j * PK + i * BP + rid
        yj = jnp.where(node < N, yj, 0.0)
        out_ref[:, j * DIM:(j + 1) * DIM] = yj


def _tc_mlp_body(y_ref, a0_ref, a1_ref, w2_ref, b1_ref, b2_ref,
                 scale_ref, shift_ref, wn_ref, out_ref):
    t = y_ref[...] + a0_ref[...] + a1_ref[...] + b1_ref[...]
    z = jnp.maximum(t, 0.0)
    z2 = jnp.dot(z, w2_ref[...], preferred_element_type=jnp.float32)
    z2 = jnp.maximum(z2 + b2_ref[...], 0.0)
    h = z2 * scale_ref[...] + shift_ref[...]
    out_ref[...] = jnp.dot(h, wn_ref[...], preferred_element_type=jnp.float32)


def _tc_mlp_pool_body(y_ref, a0_ref, a1_ref, w2_ref, b1_ref, b2_ref,
                      scale_ref, shift_ref,
                      bt0_ref, bt1_ref, bt2_ref, bt3_ref,
                      fc1w_ref, fc1b_ref, fc2w_ref, fc2b_ref,
                      out_ref, pooled_acc):
    i = pl.program_id(0)

    @pl.when(i == 0)
    def _():
        pooled_acc[...] = jnp.zeros_like(pooled_acc)

    t = y_ref[...] + a0_ref[...] + a1_ref[...] + b1_ref[...]
    z = jnp.maximum(t, 0.0)
    z2 = jnp.dot(z, w2_ref[...], preferred_element_type=jnp.float32)
    z2 = jnp.maximum(z2 + b2_ref[...], 0.0)
    h = z2 * scale_ref[...] + shift_ref[...]

    gid = lax.broadcasted_iota(jnp.int32, (G, BP), 0)
    for j, bt_ref in enumerate((bt0_ref, bt1_ref, bt2_ref, bt3_ref)):
        bj = bt_ref[0, 0, :]
        onehot = jnp.where(gid == bj[None, :], 1.0, 0.0)
        pooled_acc[...] += jnp.dot(onehot, h[:, j * DIM:(j + 1) * DIM],
                                   preferred_element_type=jnp.float32)

    @pl.when(i == pl.num_programs(0) - 1)
    def _():
        g1 = jnp.dot(pooled_acc[...], fc1w_ref[...],
                     preferred_element_type=jnp.float32)
        g1 = jnp.maximum(g1 + fc1b_ref[...], 0.0)
        out_ref[...] = (jnp.dot(g1, fc2w_ref[...],
                                preferred_element_type=jnp.float32)
                        + fc2b_ref[...])


_row_spec = pl.BlockSpec((BP, NJ * DIM), lambda i: (i, 0))
_full = lambda shape: pl.BlockSpec(shape, lambda i: tuple(0 for _ in shape))


@jax.jit
def _tc_prep(edge3d, x, w1):
    x_specs = [pl.BlockSpec((BP, F_IN), functools.partial(
        lambda j, i: (j * NBLK + i, 0), j)) for j in range(NJ)]
    e_spec = lambda r: pl.BlockSpec((1, BP, CHUNK), lambda i, r=r: (r, i, 0))
    return pl.pallas_call(
        _tc_prep_body,
        grid=(NBLK,),
        in_specs=[e_spec(0), e_spec(1)] + x_specs + [_full((F_IN, DIM))],
        out_specs=[pl.BlockSpec((BP, CHUNK), lambda i: (i, 0)),
                   pl.BlockSpec((BP, CHUNK), lambda i: (i, 0)),
                   _row_spec],
        out_shape=[jax.ShapeDtypeStruct((NW * CH_PER_W, CHUNK), jnp.int32),
                   jax.ShapeDtypeStruct((NW * CH_PER_W, CHUNK), jnp.int32),
                   jax.ShapeDtypeStruct((PK, NJ * DIM), jnp.float32)],
    )(edge3d, edge3d, x, x, x, x, w1)


@jax.jit
def _tc_mlp(y, a0, a1, w2, b1, b2, scale, shift, wn):
    vec = _full((1, NJ * DIM))
    bd = _full((NJ * DIM, NJ * DIM))
    return pl.pallas_call(
        _tc_mlp_body,
        grid=(NBLK,),
        in_specs=[_row_spec, _row_spec, _row_spec, bd, vec, vec, vec, vec, bd],
        out_specs=_row_spec,
        out_shape=jax.ShapeDtypeStruct((PK, NJ * DIM), jnp.float32),
    )(y, a0, a1, w2, b1, b2, scale, shift, wn)


@jax.jit
def _tc_mlp_pool(y, a0, a1, w2, b1, b2, scale, shift,
                 batch3d, fc1w, fc1b, fc2w, fc2b):
    vec = _full((1, NJ * DIM))
    bd = _full((NJ * DIM, NJ * DIM))
    bt_specs = [pl.BlockSpec((1, 1, BP), functools.partial(
        lambda j, i: (j * NBLK + i, 0, 0), j)) for j in range(NJ)]
    return pl.pallas_call(
        _tc_mlp_pool_body,
        grid=(NBLK,),
        in_specs=([_row_spec, _row_spec, _row_spec, bd, vec, vec, vec, vec]
                  + bt_specs
                  + [_full((DIM, DIM)), _full((1, DIM)),
                     _full((DIM, C)), _full((1, C))]),
        out_specs=_full((G, C)),
        out_shape=jax.ShapeDtypeStruct((G, C), jnp.float32),
        scratch_shapes=[pltpu.VMEM((G, DIM), jnp.float32)],
    )(y, a0, a1, w2, b1, b2, scale, shift,
      batch3d, batch3d, batch3d, batch3d, fc1w, fc1b, fc2w, fc2b)


def kernel(x, edge_index, batch,
           conv1_W1, conv1_b1, conv1_W2, conv1_b2,
           bn1_w, bn1_b, bn1_mean, bn1_var,
           conv2_W1, conv2_b1, conv2_W2, conv2_b2,
           bn2_w, bn2_b, bn2_mean, bn2_var,
           conv3_W1, conv3_b1, conv3_W2, conv3_b2,
           bn3_w, bn3_b, bn3_mean, bn3_var,
           conv4_W1, conv4_b1, conv4_W2, conv4_b2,
           bn4_w, bn4_b, bn4_mean, bn4_var,
           conv5_W1, conv5_b1, conv5_W2, conv5_b2,
           bn5_w, bn5_b, bn5_mean, bn5_var,
           fc1_W, fc1_b, fc2_W, fc2_b):
    p = locals()
    eye4 = jnp.eye(NJ, dtype=jnp.float32)
    bdiag = lambda w: jnp.kron(eye4, w)
    tile4 = lambda v: jnp.tile(v, NJ).reshape(1, NJ * DIM)

    zeros_sh = jnp.zeros((ROWS_PER_SUB, DIM), jnp.float32)
    batch_pk = jnp.concatenate([batch, jnp.full((N_SH - N,), G, jnp.int32)])
    batch3d = batch_pk.reshape(NJ * NBLK, 1, BP)

    edge3d = edge_index.reshape(2, E // CHUNK, CHUNK)
    src2d, dst2d, y = _tc_prep(edge3d, x, conv1_W1)
    for i in range(1, 6):
        agg = _sc_agg(y, src2d, dst2d, zeros_sh)
        scale = p[f"bn{i}_w"] * lax.rsqrt(p[f"bn{i}_var"] + 1e-5)
        shift = p[f"bn{i}_b"] - p[f"bn{i}_mean"] * scale
        args = (y, agg[0], agg[1], bdiag(p[f"conv{i}_W2"]),
                tile4(p[f"conv{i}_b1"]), tile4(p[f"conv{i}_b2"]),
                tile4(scale), tile4(shift))
        if i < 5:
            y = _tc_mlp(*args, bdiag(p[f"conv{i + 1}_W1"]))
        else:
            out = _tc_mlp_pool(*args, batch3d, fc1_W, fc1_b.reshape(1, -1),
                               fc2_W, fc2_b.reshape(1, -1))
    return out
